# baseline (speedup 1.0000x reference)
import functools

import jax
import jax.numpy as jnp
from jax import lax
from jax.experimental import pallas as pl
from jax.experimental.pallas import tpu as pltpu
from jax.experimental.pallas import tpu_sc as plsc

N_DIM = 4096
T_OUT = 1000000
NW = 32
B_PER_W = 31296
STRIDE_W = 31256
LAST_BASE = T_OUT - B_PER_W
N_CHUNKS = 12
B_C = B_PER_W // N_CHUNKS
UNROLL = 8


def _build_sc_kernel():
  mesh = plsc.VectorSubcoreMesh(core_axis_name="c", subcore_axis_name="s")

  idx3_t = pltpu.VMEM((3 * B_C,), jnp.int32)
  val3_t = pltpu.VMEM((3 * B_C,), jnp.float32)
  val_t = pltpu.VMEM((B_C,), jnp.float32)

  @functools.partial(
      pl.kernel,
      mesh=mesh,
      out_type=jax.ShapeDtypeStruct((T_OUT,), jnp.float32),
      scratch_types=[
          idx3_t, idx3_t,
          idx3_t, idx3_t,
          val3_t, val3_t,
          val_t, val_t,
          pltpu.SemaphoreType.DMA,
          pltpu.SemaphoreType.DMA,
          pltpu.SemaphoreType.DMA,
          pltpu.SemaphoreType.DMA,
          pltpu.SemaphoreType.DMA,
          pltpu.SemaphoreType.DMA,
          pltpu.SemaphoreType.DMA,
          pltpu.SemaphoreType.DMA,
          pltpu.SemaphoreType.DMA,
          pltpu.SemaphoreType.DMA,
      ],
  )
  def tri_gather(a_hbm, tri_hbm, out_hbm,
                 tv0, tv1, idx0, idx1, val0, val1, ov0, ov1,
                 ts00, ts01, ts02, ts10, ts11, ts12,
                 vsem0, vsem1, osem0, osem1):
    wid = lax.axis_index("s") * 2 + lax.axis_index("c")
    wbase = jnp.minimum(wid * STRIDE_W, LAST_BASE)
    tv_sets = (tv0, tv1)
    idx_sets = (idx0, idx1)
    val_sets = (val0, val1)
    tsems = ((ts00, ts01, ts02), (ts10, ts11, ts12))
    vsems = (vsem0, vsem1)
    ov_sets = (ov0, ov1)
    osems = (osem0, osem1)
    ocps = {}

    def tri_fire(c, p):
      base = wbase + c * B_C
      tvb = tv_sets[p]
      sems = tsems[p]
      return [
          pltpu.async_copy(
              tri_hbm.at[pl.ds(col * T_OUT + base, B_C)],
              tvb.at[pl.ds(col * B_C, B_C)], sems[col])
          for col in range(3)
      ]

    def compute_idx(p):
      tvb = tv_sets[p]
      idx = idx_sets[p]

      @plsc.parallel_loop(0, B_C, step=16, unroll=UNROLL)
      def _(t):
        a = tvb[pl.ds(t, 16)]
        b = tvb[pl.ds(t + B_C, 16)]
        cc = tvb[pl.ds(t + 2 * B_C, 16)]
        an = a * N_DIM
        idx[pl.ds(t, 16)] = an + b
        idx[pl.ds(t + B_C, 16)] = an + cc
        idx[pl.ds(t + 2 * B_C, 16)] = b * N_DIM + cc

    def val_fire(p):
      return pltpu.async_copy(a_hbm.at[idx_sets[p]], val_sets[p], vsems[p])

    def avg_out(c, p, cp):
      cp.wait()
      val = val_sets[p]
      ob = c % 2
      if c >= 2:
        ocps[ob].wait()
      ovb = ov_sets[ob]

      @plsc.parallel_loop(0, B_C, step=16, unroll=UNROLL)
      def _(t):
        ovb[pl.ds(t, 16)] = (
            val[pl.ds(t, 16)]
            + val[pl.ds(t + B_C, 16)]
            + val[pl.ds(t + 2 * B_C, 16)]
        ) * (1.0 / 3.0)

      base = wbase + c * B_C
      ocps[ob] = pltpu.async_copy(
          ovb, out_hbm.at[pl.ds(base, B_C)], osems[ob])

    tcps = {0: tri_fire(0, 0)}
    tcps[1] = tri_fire(1, 1)
    for h in tcps[0]:
      h.wait()
    compute_idx(0)
    vcps = {0: val_fire(0)}
    for c in range(1, N_CHUNKS):
      p = c % 2
      for h in tcps[p]:
        h.wait()
      compute_idx(p)
      vcps[p] = val_fire(p)
      if c + 1 < N_CHUNKS:
        tcps[1 - p] = tri_fire(c + 1, 1 - p)
      avg_out(c - 1, 1 - p, vcps[1 - p])
    last = N_CHUNKS - 1
    avg_out(last, last % 2, vcps[last % 2])
    ocps[0].wait()
    ocps[1].wait()

  return tri_gather


_tri_gather = _build_sc_kernel()


@jax.jit
def kernel(A_s, triangles_indexes):
  a_flat = A_s.reshape(-1)
  tri_cols = triangles_indexes.T.reshape(-1)
  return _tri_gather(a_flat, tri_cols)


# speedup vs baseline: 1.0099x; 1.0099x over previous
import functools

import jax
import jax.numpy as jnp
from jax import lax
from jax.experimental import pallas as pl
from jax.experimental.pallas import tpu as pltpu
from jax.experimental.pallas import tpu_sc as plsc

N_DIM = 4096
T_OUT = 1000000
NW = 32
B_PER_W = 31296
STRIDE_W = 31256
LAST_BASE = T_OUT - B_PER_W
N_CHUNKS = 6
B_C = B_PER_W // N_CHUNKS
UNROLL = 8


def _build_sc_kernel():
  mesh = plsc.VectorSubcoreMesh(core_axis_name="c", subcore_axis_name="s")

  idx3_t = pltpu.VMEM((3 * B_C,), jnp.int32)
  val3_t = pltpu.VMEM((3 * B_C,), jnp.float32)
  val_t = pltpu.VMEM((B_C,), jnp.float32)

  @functools.partial(
      pl.kernel,
      mesh=mesh,
      out_type=jax.ShapeDtypeStruct((T_OUT,), jnp.float32),
      scratch_types=[
          idx3_t, idx3_t,
          idx3_t, idx3_t,
          val3_t, val3_t,
          val_t, val_t,
          pltpu.SemaphoreType.DMA,
          pltpu.SemaphoreType.DMA,
          pltpu.SemaphoreType.DMA,
          pltpu.SemaphoreType.DMA,
          pltpu.SemaphoreType.DMA,
          pltpu.SemaphoreType.DMA,
          pltpu.SemaphoreType.DMA,
          pltpu.SemaphoreType.DMA,
          pltpu.SemaphoreType.DMA,
          pltpu.SemaphoreType.DMA,
          pltpu.SemaphoreType.DMA,
          pltpu.SemaphoreType.DMA,
      ],
  )
  def tri_gather(a_hbm, tri_hbm, out_hbm,
                 tv0, tv1, idx0, idx1, val0, val1, ov0, ov1,
                 ts00, ts01, ts02, ts10, ts11, ts12,
                 vsem0, vsem1, vsem2, vsem3, osem0, osem1):
    wid = lax.axis_index("s") * 2 + lax.axis_index("c")
    wbase = jnp.minimum(wid * STRIDE_W, LAST_BASE)
    tv_sets = (tv0, tv1)
    idx_sets = (idx0, idx1)
    val_sets = (val0, val1)
    tsems = ((ts00, ts01, ts02), (ts10, ts11, ts12))
    vsems = (vsem0, vsem1)
    vsems2 = (vsem2, vsem3)
    ov_sets = (ov0, ov1)
    osems = (osem0, osem1)
    ocps = {}

    def tri_fire(c, p):
      base = wbase + c * B_C
      tvb = tv_sets[p]
      sems = tsems[p]
      return [
          pltpu.async_copy(
              tri_hbm.at[pl.ds(col * T_OUT + base, B_C)],
              tvb.at[pl.ds(col * B_C, B_C)], sems[col])
          for col in range(3)
      ]

    def compute_idx(p):
      tvb = tv_sets[p]
      idx = idx_sets[p]

      @plsc.parallel_loop(0, B_C, step=16, unroll=UNROLL)
      def _(t):
        a = tvb[pl.ds(t, 16)]
        b = tvb[pl.ds(t + B_C, 16)]
        cc = tvb[pl.ds(t + 2 * B_C, 16)]
        an = a * N_DIM
        idx[pl.ds(t, 16)] = an + b
        idx[pl.ds(t + B_C, 16)] = an + cc
        idx[pl.ds(t + 2 * B_C, 16)] = b * N_DIM + cc

    H = 3 * B_C // 2

    def val_fire(p):
      idx = idx_sets[p]
      val = val_sets[p]
      return [
          pltpu.async_copy(a_hbm.at[idx.at[pl.ds(0, H)]],
                           val.at[pl.ds(0, H)], vsems[p]),
          pltpu.async_copy(a_hbm.at[idx.at[pl.ds(H, H)]],
                           val.at[pl.ds(H, H)], vsems2[p]),
      ]

    def avg_out(c, p, cp):
      for h in cp:
        h.wait()
      val = val_sets[p]
      ob = c % 2
      if c >= 2:
        ocps[ob].wait()
      ovb = ov_sets[ob]

      @plsc.parallel_loop(0, B_C, step=16, unroll=UNROLL)
      def _(t):
        ovb[pl.ds(t, 16)] = (
            val[pl.ds(t, 16)]
            + val[pl.ds(t + B_C, 16)]
            + val[pl.ds(t + 2 * B_C, 16)]
        ) * (1.0 / 3.0)

      base = wbase + c * B_C
      ocps[ob] = pltpu.async_copy(
          ovb, out_hbm.at[pl.ds(base, B_C)], osems[ob])

    tcps = {0: tri_fire(0, 0)}
    tcps[1] = tri_fire(1, 1)
    for h in tcps[0]:
      h.wait()
    compute_idx(0)
    vcps = {0: val_fire(0)}
    for c in range(1, N_CHUNKS):
      p = c % 2
      for h in tcps[p]:
        h.wait()
      compute_idx(p)
      vcps[p] = val_fire(p)
      if c + 1 < N_CHUNKS:
        tcps[1 - p] = tri_fire(c + 1, 1 - p)
      avg_out(c - 1, 1 - p, vcps[1 - p])
    last = N_CHUNKS - 1
    avg_out(last, last % 2, vcps[last % 2])
    ocps[0].wait()
    ocps[1].wait()

  return tri_gather


_tri_gather = _build_sc_kernel()


@jax.jit
def kernel(A_s, triangles_indexes):
  a_flat = A_s.reshape(-1)
  tri_cols = triangles_indexes.T.reshape(-1)
  return _tri_gather(a_flat, tri_cols)
